# Initial kernel scaffold; baseline (speedup 1.0000x reference)
import jax
import jax.numpy as jnp
from jax.experimental import pallas as pl


def kernel(x, edge_index, W1, a1s, a1d, b1, W2, a2s, a2d, b2, W3, a3s, a3d, b3):
    raise NotImplementedError("write your pallas kernel here")


# speedup vs baseline: 19.9409x; 19.9409x over previous
import functools

import jax
import jax.numpy as jnp
from jax import lax
from jax.experimental import pallas as pl
from jax.experimental.pallas import tpu as pltpu
from jax.experimental.pallas import tpu_sc as plsc

N = 10000
E = 320000
NC = 2
NS = 16
NW = NC * NS
EW = E // NW
C = 80
NCHUNK = EW // C
NP = 10240
ROWS_PER_TILE = NP // NS
DEN_ROWS = ROWS_PER_TILE // 2


def _dense_tc(x, W, P, par):
    n, k = x.shape
    f = W.shape[1]
    blk = 1000

    def body(x_ref, w_ref, p_ref, par_ref, h_ref, g_ref):
        h = jnp.dot(x_ref[...], w_ref[...], preferred_element_type=jnp.float32)
        h_ref[...] = h
        g_ref[...] = jnp.dot(h, p_ref[...],
                             preferred_element_type=jnp.float32) + par_ref[...]

    return pl.pallas_call(
        body,
        grid=(n // blk,),
        in_specs=[
            pl.BlockSpec((blk, k), lambda i: (i, 0)),
            pl.BlockSpec((k, f), lambda i: (0, 0)),
            pl.BlockSpec((f, 128), lambda i: (0, 0)),
            pl.BlockSpec((blk, 128), lambda i: (i, 0)),
        ],
        out_specs=[
            pl.BlockSpec((blk, f), lambda i: (i, 0)),
            pl.BlockSpec((blk, 128), lambda i: (i, 0)),
        ],
        out_shape=[
            jax.ShapeDtypeStruct((n, f), jnp.float32),
            jax.ShapeDtypeStruct((n, 128), jnp.float32),
        ],
    )(x, W, P, par)


def _combine_dense_tc(num, den, R, b, W, P, par):
    d1, _, f1 = num.shape
    d2, _, dw = den.shape
    f2 = W.shape[1]
    blk = 1000

    def body(num_ref, den_ref, r_ref, b_ref, w_ref, p_ref, par_ref,
             h_ref, g_ref):
        ns = jnp.sum(num_ref[...], axis=0) if d1 > 1 else num_ref[0]
        ds = jnp.sum(den_ref[...], axis=0) if d2 > 1 else den_ref[0]
        dspread = jnp.dot(ds, r_ref[...], preferred_element_type=jnp.float32)
        xx = ns / (dspread + 1e-16) + b_ref[...]
        xx = jnp.where(xx > 0, xx, jnp.exp(xx) - 1.0)
        h = jnp.dot(xx, w_ref[...], preferred_element_type=jnp.float32)
        h_ref[...] = h
        g_ref[...] = jnp.dot(h, p_ref[...],
                             preferred_element_type=jnp.float32) + par_ref[...]

    return pl.pallas_call(
        body,
        grid=(N // blk,),
        in_specs=[
            pl.BlockSpec((d1, blk, f1), lambda i: (0, i, 0)),
            pl.BlockSpec((d2, blk, dw), lambda i: (0, i, 0)),
            pl.BlockSpec((dw, f1), lambda i: (0, 0)),
            pl.BlockSpec((1, f1), lambda i: (0, 0)),
            pl.BlockSpec((f1, f2), lambda i: (0, 0)),
            pl.BlockSpec((f2, 128), lambda i: (0, 0)),
            pl.BlockSpec((blk, 128), lambda i: (i, 0)),
        ],
        out_specs=[
            pl.BlockSpec((blk, f2), lambda i: (i, 0)),
            pl.BlockSpec((blk, 128), lambda i: (i, 0)),
        ],
        out_shape=[
            jax.ShapeDtypeStruct((N, f2), jnp.float32),
            jax.ShapeDtypeStruct((N, 128), jnp.float32),
        ],
    )(num, den, R, b, W, P, par)


def _final_tc(num, den, R, b):
    f = num.shape[2]
    dw = den.shape[2]
    blk = 1000

    def body(num_ref, den_ref, r_ref, b_ref, o_ref):
        ns = num_ref[0] + num_ref[1]
        ds = den_ref[0] + den_ref[1]
        dspread = jnp.dot(ds, r_ref[...], preferred_element_type=jnp.float32)
        o_ref[...] = ns / (dspread + 1e-16) + b_ref[...]

    return pl.pallas_call(
        body,
        grid=(N // blk,),
        in_specs=[
            pl.BlockSpec((2, blk, f), lambda i: (0, i, 0)),
            pl.BlockSpec((2, blk, dw), lambda i: (0, i, 0)),
            pl.BlockSpec((dw, f), lambda i: (0, 0)),
            pl.BlockSpec((1, f), lambda i: (0, 0)),
        ],
        out_specs=pl.BlockSpec((blk, f), lambda i: (i, 0)),
        out_shape=jax.ShapeDtypeStruct((N, f), jnp.float32),
    )(num, den, R, b)


@functools.partial(jax.jit, static_argnames=("heads",))
def _edge_sc(src, dst, h, G, *, heads):
    f = 128 if heads == 8 else 16
    dshift = 1 if heads == 8 else 4
    mesh = plsc.VectorSubcoreMesh(core_axis_name="c", subcore_axis_name="s",
                                  num_cores=NC, num_subcores=NS)

    if heads == 8:
        NH = NP // 2
        DENH = NH >> dshift
        num_rows, den_rows = NH + 128, DENH + 128
        num_tile, den_tile = NH // NS, DENH // NS
        nchunk = E // NS // C
        out_type = [jax.ShapeDtypeStruct((NP, f), jnp.float32),
                    jax.ShapeDtypeStruct((NP >> dshift, 16), jnp.float32)]
    else:
        num_rows, den_rows = NP, NP >> dshift
        num_tile, den_tile = NP // NS, (NP >> dshift) // NS
        nchunk = NCHUNK
        out_type = [jax.ShapeDtypeStruct((NC, NP, f), jnp.float32),
                    jax.ShapeDtypeStruct((NC, NP >> dshift, 16), jnp.float32)]

    @functools.partial(
        pl.kernel,
        out_type=out_type,
        mesh=mesh,
        scratch_types=[
            pltpu.VMEM((C,), jnp.int32),
            pltpu.VMEM((C,), jnp.int32),
            pltpu.VMEM((C,), jnp.int32),
            pltpu.VMEM((C,), jnp.int32),
            pltpu.VMEM((C, 128), jnp.float32),
            pltpu.VMEM((C, 128), jnp.float32),
            pltpu.VMEM((C, 128), jnp.float32),
            pltpu.VMEM((C, 16), jnp.float32),
            pltpu.VMEM((C, f), jnp.float32),
            pltpu.VMEM_SHARED((num_rows, f), jnp.float32),
            pltpu.VMEM_SHARED((den_rows, 16), jnp.float32),
            pltpu.SemaphoreType.DMA,
        ],
    )
    def k(src_hbm, dst_hbm, h_hbm, g_hbm, num_hbm, den_hbm,
          sidx, didx, didxn, didxd, gs_v, gd_v, hrows_v, w_v, msg_v,
          num_s, den_s, sem):
        cid = lax.axis_index("c")
        sid = lax.axis_index("s")

        def zrow(r, cc):
            for o in range(f // 16):
                msg_v[r, pl.ds(o * 16, 16)] = jnp.zeros((16,), jnp.float32)
            w_v[r, :] = jnp.zeros((16,), jnp.float32)
            return cc
        lax.fori_loop(0, C, zrow, 0)
        for j in range(num_tile // C):
            pltpu.sync_copy(msg_v, num_s.at[pl.ds(sid * num_tile + j * C, C)])
        if den_tile >= C:
            for j in range(den_tile // C):
                pltpu.sync_copy(w_v, den_s.at[pl.ds(sid * den_tile + j * C, C)])
        else:
            pltpu.sync_copy(w_v.at[pl.ds(0, den_tile)],
                            den_s.at[pl.ds(sid * den_tile, den_tile)])
        if heads == 8:
            @pl.when(sid == 0)
            def _():
                pltpu.sync_copy(msg_v.at[pl.ds(0, C)],
                                num_s.at[pl.ds(NH, C)])
                pltpu.sync_copy(msg_v.at[pl.ds(0, 128 - C)],
                                num_s.at[pl.ds(NH + C, 128 - C)])
                pltpu.sync_copy(w_v.at[pl.ds(0, C)],
                                den_s.at[pl.ds(DENH, C)])
                pltpu.sync_copy(w_v.at[pl.ds(0, 128 - C)],
                                den_s.at[pl.ds(DENH + C, 128 - C)])
        plsc.subcore_barrier()

        if heads == 8:
            base0 = sid * (E // NS)
            nbase = cid * NH
            sel_f = jnp.where(lax.iota(jnp.int32, 16) >= 8, 1.0, 0.0)
        else:
            base0 = (sid * NC + cid) * EW
            sel_f = lax.iota(jnp.int32, 16).astype(jnp.float32)

        def chunk(i, carry):
            base = base0 + i * C
            pltpu.sync_copy(src_hbm.at[pl.ds(base, C)], sidx)
            pltpu.sync_copy(dst_hbm.at[pl.ds(base, C)], didx)
            cp1 = pltpu.async_copy(g_hbm.at[sidx], gs_v, sem)
            cp2 = pltpu.async_copy(g_hbm.at[didx], gd_v, sem)
            if heads == 8:
                cp3 = pltpu.async_copy(h_hbm.at[sidx], hrows_v, sem)
                cp3.wait()
            cp1.wait()
            cp2.wait()

            def locidx(g, cc):
                d = didx[pl.ds(g * 16, 16)]
                if heads == 8:
                    t = d - nbase
                    inr = (t >= 0) & (t < NH)
                    didxn[pl.ds(g * 16, 16)] = jnp.where(inr, t, NH)
                    didxd[pl.ds(g * 16, 16)] = jnp.where(
                        inr, lax.shift_right_logical(t, dshift), DENH)
                else:
                    didxn[pl.ds(g * 16, 16)] = d
                    didxd[pl.ds(g * 16, 16)] = lax.shift_right_logical(
                        d, dshift)
                return cc
            lax.fori_loop(0, C // 16, locidx, 0)

            def edge(c, cc):
                e = gs_v[c, pl.ds(0, 16)] + gd_v[c, pl.ds(16, 16)]
                e = jnp.where(e >= 0, e, 0.2 * e)
                w = jnp.exp(e)
                p = gd_v[c, pl.ds(48 if heads == 8 else 64, 16)]
                w_v[c, :] = jnp.where(sel_f == p, w, 0.0)
                if heads == 1:
                    msg_v[c, :] = gs_v[c, pl.ds(32, 16)] * w
                else:
                    for hh in range(heads):
                        wsp = w[jnp.full((16,), hh, jnp.int32)]
                        msg_v[c, pl.ds(hh * 16, 16)] = (
                            hrows_v[c, pl.ds(hh * 16, 16)] * wsp)
                return cc

            lax.fori_loop(0, C, edge, 0, unroll=2)
            pltpu.sync_copy(msg_v, num_s.at[didxn], add=True)
            pltpu.sync_copy(w_v, den_s.at[didxd], add=True)
            return carry

        lax.fori_loop(0, nchunk, chunk, 0)
        plsc.subcore_barrier()

        if heads == 8:
            pltpu.sync_copy(
                num_s.at[pl.ds(sid * num_tile, num_tile)],
                num_hbm.at[pl.ds(cid * NH + sid * num_tile, num_tile)])
            pltpu.sync_copy(
                den_s.at[pl.ds(sid * den_tile, den_tile)],
                den_hbm.at[pl.ds(cid * DENH + sid * den_tile, den_tile)])
        else:
            pltpu.sync_copy(
                num_s.at[pl.ds(sid * num_tile, num_tile)],
                num_hbm.at[cid, pl.ds(sid * num_tile, num_tile)])
            pltpu.sync_copy(
                den_s.at[pl.ds(sid * den_tile, den_tile)],
                den_hbm.at[cid, pl.ds(sid * den_tile, den_tile)])

    num, den = k(src, dst, h, G)
    if heads == 8:
        return num, den.reshape(NP, 8)
    return num, den.reshape(NC, NP, 16 >> dshift)


def _dup_attn(a):
    heads, ch = a.shape
    if heads == 1:
        return jnp.tile(a.reshape(ch, 1), (1, 16))
    eye = jnp.eye(heads, dtype=a.dtype)
    m = (a[:, :, None] * eye[:, None, :]).reshape(heads * ch, heads)
    return jnp.concatenate([m, m], axis=1)


def _pack_mat(a_s, a_d, f):
    cols = [_dup_attn(a_s), _dup_attn(a_d)]
    if f == 16:
        cols.append(jnp.eye(16, dtype=jnp.float32))
        cols.append(jnp.zeros((16, 128 - 48), jnp.float32))
    else:
        cols.append(jnp.zeros((f, 128 - 32), jnp.float32))
    return jnp.concatenate(cols, axis=1)


def kernel(x, edge_index, W1, a1s, a1d, b1, W2, a2s, a2d, b2, W3, a3s, a3d, b3):
    src = edge_index[0]
    dst = edge_index[1]

    P1 = _pack_mat(a1s, a1d, 128)
    P2 = _pack_mat(a2s, a2d, 16)
    P3 = _pack_mat(a3s, a3d, 16)

    R1 = jnp.repeat(jnp.eye(8, dtype=jnp.float32), 16, axis=1)
    R2 = jnp.ones((1, 16), jnp.float32)

    ids = jnp.arange(N, dtype=jnp.int32)
    par = ((ids & 1).astype(jnp.float32)[:, None]
           * jnp.zeros((1, 128), jnp.float32).at[0, 48:64].set(1.0)
           + (ids & 15).astype(jnp.float32)[:, None]
           * jnp.zeros((1, 128), jnp.float32).at[0, 64:80].set(1.0))

    h1, G1 = _dense_tc(x, W1, P1, par)
    num1, den1 = _edge_sc(src, dst, h1, G1, heads=8)
    h2, G2 = _combine_dense_tc(num1[None], den1[None], R1,
                               b1.reshape(1, 128), W2, P2, par)
    num2, den2 = _edge_sc(src, dst, h2, G2, heads=1)
    h3, G3 = _combine_dense_tc(num2, den2, R2, b2.reshape(1, 16), W3, P3, par)
    num3, den3 = _edge_sc(src, dst, h3, G3, heads=1)
    out = _final_tc(num3, den3, R2, b3.reshape(1, 16))
    return out


# speedup vs baseline: 21.2433x; 1.0653x over previous
import functools

import jax
import jax.numpy as jnp
from jax import lax
from jax.experimental import pallas as pl
from jax.experimental.pallas import tpu as pltpu
from jax.experimental.pallas import tpu_sc as plsc

N = 10000
E = 320000
NC = 2
NS = 16
NW = NC * NS
EW = E // NW
C = 80
NCHUNK = EW // C
NP = 10240
ROWS_PER_TILE = NP // NS
DEN_ROWS = ROWS_PER_TILE // 2


def _dense_tc(x, W, P, par):
    n, k = x.shape
    f = W.shape[1]
    blk = 1000

    def body(x_ref, w_ref, p_ref, par_ref, h_ref, g_ref):
        h = jnp.dot(x_ref[...], w_ref[...], preferred_element_type=jnp.float32)
        h_ref[...] = h
        g_ref[...] = jnp.dot(h, p_ref[...],
                             preferred_element_type=jnp.float32) + par_ref[...]

    return pl.pallas_call(
        body,
        grid=(n // blk,),
        in_specs=[
            pl.BlockSpec((blk, k), lambda i: (i, 0)),
            pl.BlockSpec((k, f), lambda i: (0, 0)),
            pl.BlockSpec((f, 128), lambda i: (0, 0)),
            pl.BlockSpec((blk, 128), lambda i: (i, 0)),
        ],
        out_specs=[
            pl.BlockSpec((blk, f), lambda i: (i, 0)),
            pl.BlockSpec((blk, 128), lambda i: (i, 0)),
        ],
        out_shape=[
            jax.ShapeDtypeStruct((n, f), jnp.float32),
            jax.ShapeDtypeStruct((n, 128), jnp.float32),
        ],
    )(x, W, P, par)


def _combine_dense_tc(num, den, R, b, W, P, par):
    d1, _, f1 = num.shape
    d2, _, dw = den.shape
    f2 = W.shape[1]
    blk = 1000

    def body(num_ref, den_ref, r_ref, b_ref, w_ref, p_ref, par_ref,
             h_ref, g_ref):
        ns = jnp.sum(num_ref[...], axis=0) if d1 > 1 else num_ref[0]
        ds = jnp.sum(den_ref[...], axis=0) if d2 > 1 else den_ref[0]
        dspread = jnp.dot(ds, r_ref[...], preferred_element_type=jnp.float32)
        xx = ns / (dspread + 1e-16) + b_ref[...]
        xx = jnp.where(xx > 0, xx, jnp.exp(xx) - 1.0)
        h = jnp.dot(xx, w_ref[...], preferred_element_type=jnp.float32)
        h_ref[...] = h
        g_ref[...] = jnp.dot(h, p_ref[...],
                             preferred_element_type=jnp.float32) + par_ref[...]

    return pl.pallas_call(
        body,
        grid=(N // blk,),
        in_specs=[
            pl.BlockSpec((d1, blk, f1), lambda i: (0, i, 0)),
            pl.BlockSpec((d2, blk, dw), lambda i: (0, i, 0)),
            pl.BlockSpec((dw, f1), lambda i: (0, 0)),
            pl.BlockSpec((1, f1), lambda i: (0, 0)),
            pl.BlockSpec((f1, f2), lambda i: (0, 0)),
            pl.BlockSpec((f2, 128), lambda i: (0, 0)),
            pl.BlockSpec((blk, 128), lambda i: (i, 0)),
        ],
        out_specs=[
            pl.BlockSpec((blk, f2), lambda i: (i, 0)),
            pl.BlockSpec((blk, 128), lambda i: (i, 0)),
        ],
        out_shape=[
            jax.ShapeDtypeStruct((N, f2), jnp.float32),
            jax.ShapeDtypeStruct((N, 128), jnp.float32),
        ],
    )(num, den, R, b, W, P, par)


def _final_tc(num, den, R, b):
    f = num.shape[2]
    dw = den.shape[2]
    blk = 1000

    def body(num_ref, den_ref, r_ref, b_ref, o_ref):
        ns = num_ref[0] + num_ref[1]
        ds = den_ref[0] + den_ref[1]
        dspread = jnp.dot(ds, r_ref[...], preferred_element_type=jnp.float32)
        o_ref[...] = ns / (dspread + 1e-16) + b_ref[...]

    return pl.pallas_call(
        body,
        grid=(N // blk,),
        in_specs=[
            pl.BlockSpec((2, blk, f), lambda i: (0, i, 0)),
            pl.BlockSpec((2, blk, dw), lambda i: (0, i, 0)),
            pl.BlockSpec((dw, f), lambda i: (0, 0)),
            pl.BlockSpec((1, f), lambda i: (0, 0)),
        ],
        out_specs=pl.BlockSpec((blk, f), lambda i: (i, 0)),
        out_shape=jax.ShapeDtypeStruct((N, f), jnp.float32),
    )(num, den, R, b)


@functools.partial(jax.jit, static_argnames=("heads",))
def _edge_sc(src, dst, h, G, *, heads):
    f = 128 if heads == 8 else 16
    dshift = 1 if heads == 8 else 4
    mesh = plsc.VectorSubcoreMesh(core_axis_name="c", subcore_axis_name="s",
                                  num_cores=NC, num_subcores=NS)

    if heads == 8:
        NH = NP // 2
        DENH = NH >> dshift
        num_rows, den_rows = NH + 128, DENH + 128
        num_tile, den_tile = NH // NS, DENH // NS
        nchunk = E // NS // C
        out_type = [jax.ShapeDtypeStruct((NP, f), jnp.float32),
                    jax.ShapeDtypeStruct((NP >> dshift, 16), jnp.float32)]
    else:
        num_rows, den_rows = NP, NP >> dshift
        num_tile, den_tile = NP // NS, (NP >> dshift) // NS
        nchunk = NCHUNK
        out_type = [jax.ShapeDtypeStruct((NC, NP, f), jnp.float32),
                    jax.ShapeDtypeStruct((NC, NP >> dshift, 16), jnp.float32)]

    @functools.partial(
        pl.kernel,
        out_type=out_type,
        mesh=mesh,
        scratch_types=[
            pltpu.VMEM((C,), jnp.int32),
            pltpu.VMEM((C,), jnp.int32),
            pltpu.VMEM((C,), jnp.int32),
            pltpu.VMEM((C,), jnp.int32),
            pltpu.VMEM((C, 128), jnp.float32),
            pltpu.VMEM((C, 128), jnp.float32),
            pltpu.VMEM((C, 128), jnp.float32),
            pltpu.VMEM((C, 16), jnp.float32),
            pltpu.VMEM((C, f), jnp.float32),
            pltpu.VMEM_SHARED((num_rows, f), jnp.float32),
            pltpu.VMEM_SHARED((den_rows, 16), jnp.float32),
            pltpu.SemaphoreType.DMA,
            pltpu.SemaphoreType.DMA,
        ],
    )
    def k(src_hbm, dst_hbm, h_hbm, g_hbm, num_hbm, den_hbm,
          sidx, didx, didxn, didxd, gs_v, gd_v, hrows_v, w_v, msg_v,
          num_s, den_s, sem, sem2):
        cid = lax.axis_index("c")
        sid = lax.axis_index("s")

        def zrow(r, cc):
            for o in range(f // 16):
                msg_v[r, pl.ds(o * 16, 16)] = jnp.zeros((16,), jnp.float32)
            w_v[r, :] = jnp.zeros((16,), jnp.float32)
            return cc
        lax.fori_loop(0, C, zrow, 0)
        for j in range(num_tile // C):
            pltpu.sync_copy(msg_v, num_s.at[pl.ds(sid * num_tile + j * C, C)])
        if den_tile >= C:
            for j in range(den_tile // C):
                pltpu.sync_copy(w_v, den_s.at[pl.ds(sid * den_tile + j * C, C)])
        else:
            pltpu.sync_copy(w_v.at[pl.ds(0, den_tile)],
                            den_s.at[pl.ds(sid * den_tile, den_tile)])
        if heads == 8:
            @pl.when(sid == 0)
            def _():
                pltpu.sync_copy(msg_v.at[pl.ds(0, C)],
                                num_s.at[pl.ds(NH, C)])
                pltpu.sync_copy(msg_v.at[pl.ds(0, 128 - C)],
                                num_s.at[pl.ds(NH + C, 128 - C)])
                pltpu.sync_copy(w_v.at[pl.ds(0, C)],
                                den_s.at[pl.ds(DENH, C)])
                pltpu.sync_copy(w_v.at[pl.ds(0, 128 - C)],
                                den_s.at[pl.ds(DENH + C, 128 - C)])
        plsc.subcore_barrier()

        if heads == 8:
            base0 = sid * (E // NS)
            nbase = cid * NH
            sel_f = jnp.where(lax.iota(jnp.int32, 16) >= 8, 1.0, 0.0)
        else:
            base0 = (sid * NC + cid) * EW
            sel_f = lax.iota(jnp.int32, 16).astype(jnp.float32)

        def chunk(i, carry):
            base = base0 + i * C
            pltpu.sync_copy(src_hbm.at[pl.ds(base, C)], sidx)
            pltpu.sync_copy(dst_hbm.at[pl.ds(base, C)], didx)
            cp1 = pltpu.async_copy(g_hbm.at[sidx], gs_v, sem)
            cp2 = pltpu.async_copy(g_hbm.at[didx], gd_v, sem)
            if heads == 8:
                cp3 = pltpu.async_copy(h_hbm.at[sidx], hrows_v, sem)

            @pl.when(i > 0)
            def _():
                pltpu.make_async_copy(msg_v, num_s.at[didxn], sem2).wait()
                pltpu.make_async_copy(w_v, den_s.at[didxd], sem2).wait()

            if heads == 8:
                cp3.wait()
            cp1.wait()
            cp2.wait()

            def locidx(g, cc):
                d = didx[pl.ds(g * 16, 16)]
                if heads == 8:
                    t = d - nbase
                    inr = (t >= 0) & (t < NH)
                    didxn[pl.ds(g * 16, 16)] = jnp.where(inr, t, NH)
                    didxd[pl.ds(g * 16, 16)] = jnp.where(
                        inr, lax.shift_right_logical(t, dshift), DENH)
                else:
                    didxn[pl.ds(g * 16, 16)] = d
                    didxd[pl.ds(g * 16, 16)] = lax.shift_right_logical(
                        d, dshift)
                return cc
            lax.fori_loop(0, C // 16, locidx, 0)

            def edge(c, cc):
                e = gs_v[c, pl.ds(0, 16)] + gd_v[c, pl.ds(16, 16)]
                e = jnp.where(e >= 0, e, 0.2 * e)
                w = jnp.exp(e)
                p = gd_v[c, pl.ds(48 if heads == 8 else 64, 16)]
                w_v[c, :] = jnp.where(sel_f == p, w, 0.0)
                if heads == 1:
                    msg_v[c, :] = gs_v[c, pl.ds(32, 16)] * w
                else:
                    for hh in range(heads):
                        wsp = w[jnp.full((16,), hh, jnp.int32)]
                        msg_v[c, pl.ds(hh * 16, 16)] = (
                            hrows_v[c, pl.ds(hh * 16, 16)] * wsp)
                return cc

            lax.fori_loop(0, C, edge, 0, unroll=2)
            pltpu.async_copy(msg_v, num_s.at[didxn], sem2, add=True)
            pltpu.async_copy(w_v, den_s.at[didxd], sem2, add=True)
            return carry

        lax.fori_loop(0, nchunk, chunk, 0)
        pltpu.make_async_copy(msg_v, num_s.at[didxn], sem2).wait()
        pltpu.make_async_copy(w_v, den_s.at[didxd], sem2).wait()
        plsc.subcore_barrier()

        if heads == 8:
            pltpu.sync_copy(
                num_s.at[pl.ds(sid * num_tile, num_tile)],
                num_hbm.at[pl.ds(cid * NH + sid * num_tile, num_tile)])
            pltpu.sync_copy(
                den_s.at[pl.ds(sid * den_tile, den_tile)],
                den_hbm.at[pl.ds(cid * DENH + sid * den_tile, den_tile)])
        else:
            pltpu.sync_copy(
                num_s.at[pl.ds(sid * num_tile, num_tile)],
                num_hbm.at[cid, pl.ds(sid * num_tile, num_tile)])
            pltpu.sync_copy(
                den_s.at[pl.ds(sid * den_tile, den_tile)],
                den_hbm.at[cid, pl.ds(sid * den_tile, den_tile)])

    num, den = k(src, dst, h, G)
    if heads == 8:
        return num, den.reshape(NP, 8)
    return num, den.reshape(NC, NP, 16 >> dshift)


def _dup_attn(a):
    heads, ch = a.shape
    if heads == 1:
        return jnp.tile(a.reshape(ch, 1), (1, 16))
    eye = jnp.eye(heads, dtype=a.dtype)
    m = (a[:, :, None] * eye[:, None, :]).reshape(heads * ch, heads)
    return jnp.concatenate([m, m], axis=1)


def _pack_mat(a_s, a_d, f):
    cols = [_dup_attn(a_s), _dup_attn(a_d)]
    if f == 16:
        cols.append(jnp.eye(16, dtype=jnp.float32))
        cols.append(jnp.zeros((16, 128 - 48), jnp.float32))
    else:
        cols.append(jnp.zeros((f, 128 - 32), jnp.float32))
    return jnp.concatenate(cols, axis=1)


def kernel(x, edge_index, W1, a1s, a1d, b1, W2, a2s, a2d, b2, W3, a3s, a3d, b3):
    src = edge_index[0]
    dst = edge_index[1]

    P1 = _pack_mat(a1s, a1d, 128)
    P2 = _pack_mat(a2s, a2d, 16)
    P3 = _pack_mat(a3s, a3d, 16)

    R1 = jnp.repeat(jnp.eye(8, dtype=jnp.float32), 16, axis=1)
    R2 = jnp.ones((1, 16), jnp.float32)

    ids = jnp.arange(N, dtype=jnp.int32)
    par = ((ids & 1).astype(jnp.float32)[:, None]
           * jnp.zeros((1, 128), jnp.float32).at[0, 48:64].set(1.0)
           + (ids & 15).astype(jnp.float32)[:, None]
           * jnp.zeros((1, 128), jnp.float32).at[0, 64:80].set(1.0))

    h1, G1 = _dense_tc(x, W1, P1, par)
    num1, den1 = _edge_sc(src, dst, h1, G1, heads=8)
    h2, G2 = _combine_dense_tc(num1[None], den1[None], R1,
                               b1.reshape(1, 128), W2, P2, par)
    num2, den2 = _edge_sc(src, dst, h2, G2, heads=1)
    h3, G3 = _combine_dense_tc(num2, den2, R2, b2.reshape(1, 16), W3, P3, par)
    num3, den3 = _edge_sc(src, dst, h3, G3, heads=1)
    out = _final_tc(num3, den3, R2, b3.reshape(1, 16))
    return out


# speedup vs baseline: 21.3631x; 1.0056x over previous
import functools

import jax
import jax.numpy as jnp
from jax import lax
from jax.experimental import pallas as pl
from jax.experimental.pallas import tpu as pltpu
from jax.experimental.pallas import tpu_sc as plsc

N = 10000
E = 320000
NC = 2
NS = 16
NW = NC * NS
EW = E // NW
C = 80
NCHUNK = EW // C
NP = 10240
ROWS_PER_TILE = NP // NS
DEN_ROWS = ROWS_PER_TILE // 2


def _dense_tc(x, W, P, par):
    n, k = x.shape
    f = W.shape[1]
    blk = 1000

    def body(x_ref, w_ref, p_ref, par_ref, h_ref, g_ref):
        h = jnp.dot(x_ref[...], w_ref[...], preferred_element_type=jnp.float32)
        h_ref[...] = h
        g_ref[...] = jnp.dot(h, p_ref[...],
                             preferred_element_type=jnp.float32) + par_ref[...]

    return pl.pallas_call(
        body,
        grid=(n // blk,),
        in_specs=[
            pl.BlockSpec((blk, k), lambda i: (i, 0)),
            pl.BlockSpec((k, f), lambda i: (0, 0)),
            pl.BlockSpec((f, 128), lambda i: (0, 0)),
            pl.BlockSpec((blk, 128), lambda i: (i, 0)),
        ],
        out_specs=[
            pl.BlockSpec((blk, f), lambda i: (i, 0)),
            pl.BlockSpec((blk, 128), lambda i: (i, 0)),
        ],
        out_shape=[
            jax.ShapeDtypeStruct((n, f), jnp.float32),
            jax.ShapeDtypeStruct((n, 128), jnp.float32),
        ],
    )(x, W, P, par)


def _combine_dense_tc(num, den, R, b, W, P, par):
    d1, _, f1 = num.shape
    d2, _, dw = den.shape
    f2 = W.shape[1]
    blk = 1000

    def body(num_ref, den_ref, r_ref, b_ref, w_ref, p_ref, par_ref,
             h_ref, g_ref):
        ns = jnp.sum(num_ref[...], axis=0) if d1 > 1 else num_ref[0]
        ds = jnp.sum(den_ref[...], axis=0) if d2 > 1 else den_ref[0]
        dspread = jnp.dot(ds, r_ref[...], preferred_element_type=jnp.float32)
        xx = ns / (dspread + 1e-16) + b_ref[...]
        xx = jnp.where(xx > 0, xx, jnp.exp(xx) - 1.0)
        h = jnp.dot(xx, w_ref[...], preferred_element_type=jnp.float32)
        h_ref[...] = h
        g_ref[...] = jnp.dot(h, p_ref[...],
                             preferred_element_type=jnp.float32) + par_ref[...]

    return pl.pallas_call(
        body,
        grid=(N // blk,),
        in_specs=[
            pl.BlockSpec((d1, blk, f1), lambda i: (0, i, 0)),
            pl.BlockSpec((d2, blk, dw), lambda i: (0, i, 0)),
            pl.BlockSpec((dw, f1), lambda i: (0, 0)),
            pl.BlockSpec((1, f1), lambda i: (0, 0)),
            pl.BlockSpec((f1, f2), lambda i: (0, 0)),
            pl.BlockSpec((f2, 128), lambda i: (0, 0)),
            pl.BlockSpec((blk, 128), lambda i: (i, 0)),
        ],
        out_specs=[
            pl.BlockSpec((blk, f2), lambda i: (i, 0)),
            pl.BlockSpec((blk, 128), lambda i: (i, 0)),
        ],
        out_shape=[
            jax.ShapeDtypeStruct((N, f2), jnp.float32),
            jax.ShapeDtypeStruct((N, 128), jnp.float32),
        ],
    )(num, den, R, b, W, P, par)


def _final_tc(num, den, R, b):
    f = num.shape[2]
    dw = den.shape[2]
    blk = 1000

    def body(num_ref, den_ref, r_ref, b_ref, o_ref):
        ns = num_ref[0] + num_ref[1]
        ds = den_ref[0] + den_ref[1]
        dspread = jnp.dot(ds, r_ref[...], preferred_element_type=jnp.float32)
        o_ref[...] = ns / (dspread + 1e-16) + b_ref[...]

    return pl.pallas_call(
        body,
        grid=(N // blk,),
        in_specs=[
            pl.BlockSpec((2, blk, f), lambda i: (0, i, 0)),
            pl.BlockSpec((2, blk, dw), lambda i: (0, i, 0)),
            pl.BlockSpec((dw, f), lambda i: (0, 0)),
            pl.BlockSpec((1, f), lambda i: (0, 0)),
        ],
        out_specs=pl.BlockSpec((blk, f), lambda i: (i, 0)),
        out_shape=jax.ShapeDtypeStruct((N, f), jnp.float32),
    )(num, den, R, b)


@functools.partial(jax.jit, static_argnames=("heads",))
def _edge_sc(src, dst, h, G, *, heads):
    f = 128 if heads == 8 else 16
    dshift = 1 if heads == 8 else 4
    mesh = plsc.VectorSubcoreMesh(core_axis_name="c", subcore_axis_name="s",
                                  num_cores=NC, num_subcores=NS)

    if heads == 8:
        NH = NP // 2
        DENH = NH >> dshift
        num_rows, den_rows = NH + 128, DENH + 128
        num_tile, den_tile = NH // NS, DENH // NS
        nchunk = E // NS // C
        out_type = [jax.ShapeDtypeStruct((NP, f), jnp.float32),
                    jax.ShapeDtypeStruct((NP >> dshift, 16), jnp.float32)]
    else:
        num_rows, den_rows = NP, NP >> dshift
        num_tile, den_tile = NP // NS, (NP >> dshift) // NS
        nchunk = NCHUNK
        out_type = [jax.ShapeDtypeStruct((NC, NP, f), jnp.float32),
                    jax.ShapeDtypeStruct((NC, NP >> dshift, 16), jnp.float32)]

    @functools.partial(
        pl.kernel,
        out_type=out_type,
        mesh=mesh,
        scratch_types=[
            pltpu.VMEM((C,), jnp.int32),
            pltpu.VMEM((C,), jnp.int32),
            pltpu.VMEM((C,), jnp.int32),
            pltpu.VMEM((C,), jnp.int32),
            pltpu.VMEM((C, 128), jnp.float32),
            pltpu.VMEM((C, 128), jnp.float32),
            pltpu.VMEM((C, 128), jnp.float32),
            pltpu.VMEM((C, 16), jnp.float32),
            pltpu.VMEM((C, f), jnp.float32),
            pltpu.VMEM_SHARED((num_rows, f), jnp.float32),
            pltpu.VMEM_SHARED((den_rows, 16), jnp.float32),
            pltpu.SemaphoreType.DMA,
            pltpu.SemaphoreType.DMA,
        ],
    )
    def k(src_hbm, dst_hbm, h_hbm, g_hbm, num_hbm, den_hbm,
          sidx, didx, didxn, didxd, gs_v, gd_v, hrows_v, w_v, msg_v,
          num_s, den_s, sem, sem2):
        cid = lax.axis_index("c")
        sid = lax.axis_index("s")

        def zrow(r, cc):
            for o in range(f // 16):
                msg_v[r, pl.ds(o * 16, 16)] = jnp.zeros((16,), jnp.float32)
            w_v[r, :] = jnp.zeros((16,), jnp.float32)
            return cc
        lax.fori_loop(0, C, zrow, 0)
        for j in range(num_tile // C):
            pltpu.sync_copy(msg_v, num_s.at[pl.ds(sid * num_tile + j * C, C)])
        if den_tile >= C:
            for j in range(den_tile // C):
                pltpu.sync_copy(w_v, den_s.at[pl.ds(sid * den_tile + j * C, C)])
        else:
            pltpu.sync_copy(w_v.at[pl.ds(0, den_tile)],
                            den_s.at[pl.ds(sid * den_tile, den_tile)])
        if heads == 8:
            @pl.when(sid == 0)
            def _():
                pltpu.sync_copy(msg_v.at[pl.ds(0, C)],
                                num_s.at[pl.ds(NH, C)])
                pltpu.sync_copy(msg_v.at[pl.ds(0, 128 - C)],
                                num_s.at[pl.ds(NH + C, 128 - C)])
                pltpu.sync_copy(w_v.at[pl.ds(0, C)],
                                den_s.at[pl.ds(DENH, C)])
                pltpu.sync_copy(w_v.at[pl.ds(0, 128 - C)],
                                den_s.at[pl.ds(DENH + C, 128 - C)])
        plsc.subcore_barrier()

        if heads == 8:
            base0 = sid * (E // NS)
            nbase = cid * NH
            sel_f = jnp.where(lax.iota(jnp.int32, 16) >= 8, 1.0, 0.0)
        else:
            base0 = (sid * NC + cid) * EW
            sel_f = lax.iota(jnp.int32, 16).astype(jnp.float32)

        def chunk(i, carry):
            base = base0 + i * C
            pltpu.sync_copy(src_hbm.at[pl.ds(base, C)], sidx)
            pltpu.sync_copy(dst_hbm.at[pl.ds(base, C)], didx)
            cp1 = pltpu.async_copy(g_hbm.at[sidx], gs_v, sem)
            cp2 = pltpu.async_copy(g_hbm.at[didx], gd_v, sem)
            if heads == 8:
                cp3 = pltpu.async_copy(h_hbm.at[sidx], hrows_v, sem)

            @pl.when(i > 0)
            def _():
                pltpu.make_async_copy(msg_v, num_s.at[didxn], sem2).wait()
                pltpu.make_async_copy(w_v, den_s.at[didxd], sem2).wait()

            if heads == 8:
                cp3.wait()
            cp1.wait()
            cp2.wait()

            def locidx(g, cc):
                d = didx[pl.ds(g * 16, 16)]
                if heads == 8:
                    t = d - nbase
                    inr = (t >= 0) & (t < NH)
                    didxn[pl.ds(g * 16, 16)] = jnp.where(inr, t, NH)
                    didxd[pl.ds(g * 16, 16)] = jnp.where(
                        inr, lax.shift_right_logical(t, dshift), DENH)
                else:
                    didxn[pl.ds(g * 16, 16)] = d
                    didxd[pl.ds(g * 16, 16)] = lax.shift_right_logical(
                        d, dshift)
                return cc
            lax.fori_loop(0, C // 16, locidx, 0)

            def edge(c, cc):
                e = gs_v[c, pl.ds(0, 16)] + gd_v[c, pl.ds(16, 16)]
                e = jnp.where(e >= 0, e, 0.2 * e)
                w = jnp.exp(e)
                p = gd_v[c, pl.ds(48 if heads == 8 else 64, 16)]
                w_v[c, :] = jnp.where(sel_f == p, w, 0.0)
                if heads == 1:
                    msg_v[c, :] = gs_v[c, pl.ds(32, 16)] * w
                else:
                    for hh in range(heads):
                        wsp = w[jnp.full((16,), hh, jnp.int32)]
                        msg_v[c, pl.ds(hh * 16, 16)] = (
                            hrows_v[c, pl.ds(hh * 16, 16)] * wsp)
                return cc

            lax.fori_loop(0, C, edge, 0, unroll=4)
            pltpu.async_copy(msg_v, num_s.at[didxn], sem2, add=True)
            pltpu.async_copy(w_v, den_s.at[didxd], sem2, add=True)
            return carry

        lax.fori_loop(0, nchunk, chunk, 0)
        pltpu.make_async_copy(msg_v, num_s.at[didxn], sem2).wait()
        pltpu.make_async_copy(w_v, den_s.at[didxd], sem2).wait()
        plsc.subcore_barrier()

        if heads == 8:
            pltpu.sync_copy(
                num_s.at[pl.ds(sid * num_tile, num_tile)],
                num_hbm.at[pl.ds(cid * NH + sid * num_tile, num_tile)])
            pltpu.sync_copy(
                den_s.at[pl.ds(sid * den_tile, den_tile)],
                den_hbm.at[pl.ds(cid * DENH + sid * den_tile, den_tile)])
        else:
            pltpu.sync_copy(
                num_s.at[pl.ds(sid * num_tile, num_tile)],
                num_hbm.at[cid, pl.ds(sid * num_tile, num_tile)])
            pltpu.sync_copy(
                den_s.at[pl.ds(sid * den_tile, den_tile)],
                den_hbm.at[cid, pl.ds(sid * den_tile, den_tile)])

    num, den = k(src, dst, h, G)
    if heads == 8:
        return num, den.reshape(NP, 8)
    return num, den.reshape(NC, NP, 16 >> dshift)


def _dup_attn(a):
    heads, ch = a.shape
    if heads == 1:
        return jnp.tile(a.reshape(ch, 1), (1, 16))
    eye = jnp.eye(heads, dtype=a.dtype)
    m = (a[:, :, None] * eye[:, None, :]).reshape(heads * ch, heads)
    return jnp.concatenate([m, m], axis=1)


def _pack_mat(a_s, a_d, f):
    cols = [_dup_attn(a_s), _dup_attn(a_d)]
    if f == 16:
        cols.append(jnp.eye(16, dtype=jnp.float32))
        cols.append(jnp.zeros((16, 128 - 48), jnp.float32))
    else:
        cols.append(jnp.zeros((f, 128 - 32), jnp.float32))
    return jnp.concatenate(cols, axis=1)


def kernel(x, edge_index, W1, a1s, a1d, b1, W2, a2s, a2d, b2, W3, a3s, a3d, b3):
    src = edge_index[0]
    dst = edge_index[1]

    P1 = _pack_mat(a1s, a1d, 128)
    P2 = _pack_mat(a2s, a2d, 16)
    P3 = _pack_mat(a3s, a3d, 16)

    R1 = jnp.repeat(jnp.eye(8, dtype=jnp.float32), 16, axis=1)
    R2 = jnp.ones((1, 16), jnp.float32)

    ids = jnp.arange(N, dtype=jnp.int32)
    par = ((ids & 1).astype(jnp.float32)[:, None]
           * jnp.zeros((1, 128), jnp.float32).at[0, 48:64].set(1.0)
           + (ids & 15).astype(jnp.float32)[:, None]
           * jnp.zeros((1, 128), jnp.float32).at[0, 64:80].set(1.0))

    h1, G1 = _dense_tc(x, W1, P1, par)
    num1, den1 = _edge_sc(src, dst, h1, G1, heads=8)
    h2, G2 = _combine_dense_tc(num1[None], den1[None], R1,
                               b1.reshape(1, 128), W2, P2, par)
    num2, den2 = _edge_sc(src, dst, h2, G2, heads=1)
    h3, G3 = _combine_dense_tc(num2, den2, R2, b2.reshape(1, 16), W3, P3, par)
    num3, den3 = _edge_sc(src, dst, h3, G3, heads=1)
    out = _final_tc(num3, den3, R2, b3.reshape(1, 16))
    return out
